# Initial kernel scaffold; baseline (speedup 1.0000x reference)
import jax
import jax.numpy as jnp
from jax.experimental import pallas as pl


def kernel(q, k, v, edge_index, Wq, Wk, Wv, Wfc, W1, as1, ad1, b1, Wl1, bl1, W2, as2, ad2, b2, Wl2, bl2, W3, as3, ad3, b3, Wl3, bl3, gamma, beta):
    raise NotImplementedError("write your pallas kernel here")


# speedup vs baseline: 294.4137x; 294.4137x over previous
import functools

import jax
import jax.numpy as jnp
from jax import lax
from jax.experimental import pallas as pl
from jax.experimental.pallas import tpu as pltpu
from jax.experimental.pallas import tpu_sc as plsc

BS, N, D_MODEL = 16, 512, 128
E = 8192
HEADS = 2
D_K = 64
PH = 256

NC, NS, L = 2, 16, 16
NW = NC * NS
ROWS = 128
NBLK = N // ROWS
NASSIGN = BS * NBLK
BLKW = ROWS * N


def _sc_count_kernel(src_hbm, dst_hbm, out_hbm, blk, src_v, dst_v):
    cid = lax.axis_index("c")
    sid = lax.axis_index("s")
    wid = sid * NC + cid

    ones = jnp.ones((L,), jnp.float32)
    zeros = jnp.zeros((L,), jnp.float32)

    for r in range(NASSIGN // NW):
        aid = wid + NW * r
        b = aid // NBLK
        lo = (aid % NBLK) * ROWS

        pltpu.sync_copy(src_hbm.at[b], src_v)
        pltpu.sync_copy(dst_hbm.at[b], dst_v)

        def zero_body(i, _):
            blk[pl.ds(i * L, L)] = zeros
            return 0
        lax.fori_loop(0, BLKW // L, zero_body, 0, unroll=8)

        def edge_body(i, _):
            s = src_v[pl.ds(i * L, L)]
            d = dst_v[pl.ds(i * L, L)]
            row = d - lo
            m = (row >= 0) & (row < ROWS)
            idx = jnp.where(m, row * N + s, 0)
            plsc.addupdate_scatter(blk, [idx], ones, mask=m)
            return 0
        lax.fori_loop(0, E // L, edge_body, 0, unroll=4)

        pltpu.sync_copy(blk, out_hbm.at[aid])


def _build_counts(src, dst):
    mesh = plsc.VectorSubcoreMesh(
        core_axis_name="c", subcore_axis_name="s", num_cores=NC, num_subcores=NS
    )
    counts = pl.kernel(
        _sc_count_kernel,
        out_type=jax.ShapeDtypeStruct((NASSIGN, BLKW), jnp.float32),
        mesh=mesh,
        scratch_types=[
            pltpu.VMEM((BLKW,), jnp.float32),
            pltpu.VMEM((E,), jnp.int32),
            pltpu.VMEM((E,), jnp.int32),
        ],
        compiler_params=pltpu.CompilerParams(needs_layout_passes=False),
    )(src, dst)
    return counts.reshape(BS, N, N)


def _mm(a, b):
    return lax.dot_general(a, b, (((1,), (0,)), ((), ())),
                           preferred_element_type=jnp.float32)


def _mm_t(a, b):
    return lax.dot_general(a, b, (((1,), (1,)), ((), ())),
                           preferred_element_type=jnp.float32)


def _gat_head(xp_h, a_s_h, a_d_h, cpos, cnt):
    al_s = _mm_t(xp_h, a_s_h)
    al_d = _mm_t(xp_h, a_d_h)
    e = al_d + al_s.reshape(1, N)
    e = jnp.where(e > 0.0, e, 0.2 * e)
    masked = jnp.where(cpos, e, -1e30)
    m = jnp.max(masked, axis=1, keepdims=True)
    w = cnt * jnp.exp(jnp.minimum(e - m, 0.0))
    ssum = jnp.sum(w, axis=1, keepdims=True)
    alpha = w / (ssum + 1e-16)
    return _mm(alpha, xp_h)


def _tc_body(c_ref, q_ref, k_ref, v_ref,
             wq_ref, wk_ref, wv_ref, wfc_ref,
             w1_ref, as1_ref, ad1_ref, b1_ref, wl1_ref, bl1_ref,
             w2_ref, as2_ref, ad2_ref, b2_ref, wl2_ref, bl2_ref,
             w3_ref, as3_ref, ad3_ref, b3_ref, wl3_ref, bl3_ref,
             gamma_ref, beta_ref,
             out_ref, attn_ref):
    cnt = c_ref[0]
    cpos = cnt > 0.0
    x = q_ref[0]

    def gat_layer(h, w_ref, as_ref, ad_ref, b_ref, od, concat):
        xp = _mm(h, w_ref[...])
        outs = []
        for hd in range(HEADS):
            xp_h = xp[:, hd * od:(hd + 1) * od]
            a_s = as_ref[hd:hd + 1, :]
            a_d = ad_ref[hd:hd + 1, :]
            outs.append(_gat_head(xp_h, a_s, a_d, cpos, cnt))
        if concat:
            o = jnp.concatenate(outs, axis=1)
        else:
            o = (outs[0] + outs[1]) * 0.5
        return o + b_ref[...]

    def elu(z):
        return jnp.where(z > 0.0, z, jnp.exp(jnp.minimum(z, 0.0)) - 1.0)

    h1 = elu(gat_layer(x, w1_ref, as1_ref, ad1_ref, b1_ref, PH, True)
             + _mm(x, wl1_ref[...]) + bl1_ref[...])
    h2 = elu(gat_layer(h1, w2_ref, as2_ref, ad2_ref, b2_ref, PH, True)
             + _mm(h1, wl2_ref[...]) + bl2_ref[...])
    x3 = (gat_layer(h2, w3_ref, as3_ref, ad3_ref, b3_ref, 2 * D_K, False)
          + _mm(h2, wl3_ref[...]) + bl3_ref[...])

    qh = _mm(x, wq_ref[...])
    kh = _mm(k_ref[0], wk_ref[...])
    vh = _mm(v_ref[0], wv_ref[...])
    scale = 1.0 / (D_K ** 0.5)
    os = []
    for hd in range(HEADS):
        q_h = qh[:, hd * D_K:(hd + 1) * D_K] * scale
        k_h = kh[:, hd * D_K:(hd + 1) * D_K]
        v_h = vh[:, hd * D_K:(hd + 1) * D_K]
        logits = _mm_t(q_h, k_h)
        mx = jnp.max(logits, axis=1, keepdims=True)
        ex = jnp.exp(logits - mx)
        a = ex / jnp.sum(ex, axis=1, keepdims=True)
        attn_ref[0, hd] = a
        os.append(_mm(a, v_h))
    o = jnp.concatenate(os, axis=1)

    wfc = wfc_ref[...]
    out = (_mm(x3, wfc[:2 * D_K, :]) + _mm(o, wfc[2 * D_K:, :]) + x)
    mu = jnp.mean(out, axis=1, keepdims=True)
    cen = out - mu
    var = jnp.mean(cen * cen, axis=1, keepdims=True)
    out_ref[0] = cen * jax.lax.rsqrt(var + 1e-6) * gamma_ref[...] + beta_ref[...]


def _tc_forward(C, q, k, v, Wq, Wk, Wv, Wfc,
                W1, as1, ad1, b1, Wl1, bl1,
                W2, as2, ad2, b2, Wl2, bl2,
                W3, as3, ad3, b3, Wl3, bl3,
                gamma, beta, interpret=False):
    full = lambda shape: pl.BlockSpec(shape, lambda b: (0,) * len(shape))
    grid_spec = pl.GridSpec(
        grid=(BS,),
        in_specs=[
            pl.BlockSpec((1, N, N), lambda b: (b, 0, 0)),
            pl.BlockSpec((1, N, D_MODEL), lambda b: (b, 0, 0)),
            pl.BlockSpec((1, N, D_MODEL), lambda b: (b, 0, 0)),
            pl.BlockSpec((1, N, D_MODEL), lambda b: (b, 0, 0)),
            full((D_MODEL, HEADS * D_K)),
            full((D_MODEL, HEADS * D_K)),
            full((D_MODEL, HEADS * D_K)),
            full((4 * D_K, D_MODEL)),
            full((D_MODEL, 2 * PH)),
            full((2, PH)), full((2, PH)), full((1, 2 * PH)),
            full((D_MODEL, 2 * PH)), full((1, 2 * PH)),
            full((2 * PH, 2 * PH)),
            full((2, PH)), full((2, PH)), full((1, 2 * PH)),
            full((2 * PH, 2 * PH)), full((1, 2 * PH)),
            full((2 * PH, 2 * 2 * D_K)),
            full((2, 2 * D_K)), full((2, 2 * D_K)), full((1, 2 * D_K)),
            full((2 * PH, 2 * D_K)), full((1, 2 * D_K)),
            full((1, D_MODEL)), full((1, D_MODEL)),
        ],
        out_specs=[
            pl.BlockSpec((1, N, D_MODEL), lambda b: (b, 0, 0)),
            pl.BlockSpec((1, HEADS, N, N), lambda b: (b, 0, 0, 0)),
        ],
    )
    return pl.pallas_call(
        _tc_body,
        grid_spec=grid_spec,
        out_shape=[
            jax.ShapeDtypeStruct((BS, N, D_MODEL), jnp.float32),
            jax.ShapeDtypeStruct((BS, HEADS, N, N), jnp.float32),
        ],
        interpret=interpret,
    )(C, q, k, v, Wq, Wk, Wv, Wfc,
      W1, as1, ad1, b1.reshape(1, -1), Wl1, bl1.reshape(1, -1),
      W2, as2, ad2, b2.reshape(1, -1), Wl2, bl2.reshape(1, -1),
      W3, as3, ad3, b3.reshape(1, -1), Wl3, bl3.reshape(1, -1),
      gamma.reshape(1, -1), beta.reshape(1, -1))


def kernel(q, k, v, edge_index, Wq, Wk, Wv, Wfc, W1, as1, ad1, b1, Wl1, bl1,
           W2, as2, ad2, b2, Wl2, bl2, W3, as3, ad3, b3, Wl3, bl3,
           gamma, beta):
    src = edge_index[:, 0, :]
    dst = edge_index[:, 1, :]
    C = _build_counts(src, dst)
    out, attn = _tc_forward(C, q, k, v, Wq, Wk, Wv, Wfc,
                            W1, as1, ad1, b1, Wl1, bl1,
                            W2, as2, ad2, b2, Wl2, bl2,
                            W3, as3, ad3, b3, Wl3, bl3,
                            gamma, beta)
    return (out, attn)


# speedup vs baseline: 315.8676x; 1.0729x over previous
import functools

import jax
import jax.numpy as jnp
from jax import lax
from jax.experimental import pallas as pl
from jax.experimental.pallas import tpu as pltpu
from jax.experimental.pallas import tpu_sc as plsc

BS, N, D_MODEL = 16, 512, 128
E = 8192
HEADS = 2
D_K = 64
PH = 256

NC, NS, L = 2, 16, 16
NW = NC * NS
ROWS = 128
NBLK = N // ROWS
NASSIGN = BS * NBLK
BLKW = ROWS * N


def _sc_count_kernel(src_hbm, dst_hbm, out_hbm, blk, src_v, dst_v):
    cid = lax.axis_index("c")
    sid = lax.axis_index("s")
    wid = sid * NC + cid

    ones = jnp.ones((L,), jnp.float32)
    zeros = jnp.zeros((L,), jnp.float32)

    for r in range(NASSIGN // NW):
        aid = wid + NW * r
        b = aid // NBLK
        lo = (aid % NBLK) * ROWS

        pltpu.sync_copy(src_hbm.at[b], src_v)
        pltpu.sync_copy(dst_hbm.at[b], dst_v)

        def zero_body(i, _):
            blk[pl.ds(i * L, L)] = zeros
            return 0
        lax.fori_loop(0, BLKW // L, zero_body, 0, unroll=8)

        def edge_body(i, _):
            s = src_v[pl.ds(i * L, L)]
            d = dst_v[pl.ds(i * L, L)]
            row = d - lo
            m = (row >= 0) & (row < ROWS)
            idx = jnp.where(m, row * N + s, 0)
            plsc.addupdate_scatter(blk, [idx], ones, mask=m)
            return 0
        lax.fori_loop(0, E // L, edge_body, 0, unroll=4)

        pltpu.sync_copy(blk, out_hbm.at[aid])


def _build_counts(src, dst):
    mesh = plsc.VectorSubcoreMesh(
        core_axis_name="c", subcore_axis_name="s", num_cores=NC, num_subcores=NS
    )
    counts = pl.kernel(
        _sc_count_kernel,
        out_type=jax.ShapeDtypeStruct((NASSIGN, BLKW), jnp.float32),
        mesh=mesh,
        scratch_types=[
            pltpu.VMEM((BLKW,), jnp.float32),
            pltpu.VMEM((E,), jnp.int32),
            pltpu.VMEM((E,), jnp.int32),
        ],
        compiler_params=pltpu.CompilerParams(needs_layout_passes=False),
    )(src, dst)
    return counts.reshape(BS, N, N)


def _mm(a, b):
    return lax.dot_general(a, b, (((1,), (0,)), ((), ())),
                           preferred_element_type=jnp.float32)


def _mm_t(a, b):
    return lax.dot_general(a, b, (((1,), (1,)), ((), ())),
                           preferred_element_type=jnp.float32)


def _gat_head(xp_h, a_s_h, a_d_h, cpos, cnt):
    al_s = _mm_t(xp_h, a_s_h)
    al_d = _mm_t(xp_h, a_d_h)
    e = al_d + al_s.reshape(1, N)
    e = jnp.where(e > 0.0, e, 0.2 * e)
    masked = jnp.where(cpos, e, -1e30)
    m = jnp.max(masked, axis=1, keepdims=True)
    w = cnt * jnp.exp(masked - m)
    ssum = jnp.sum(w, axis=1, keepdims=True)
    return _mm(w, xp_h) * (1.0 / (ssum + 1e-16))


def _tc_body(c_ref, q_ref, k_ref, v_ref,
             wq_ref, wk_ref, wv_ref, wfc_ref,
             w1_ref, as1_ref, ad1_ref, b1_ref, wl1_ref, bl1_ref,
             w2_ref, as2_ref, ad2_ref, b2_ref, wl2_ref, bl2_ref,
             w3_ref, as3_ref, ad3_ref, b3_ref, wl3_ref, bl3_ref,
             gamma_ref, beta_ref,
             out_ref, attn_ref):
    cnt = c_ref[0]
    cpos = cnt > 0.0
    x = q_ref[0]

    def gat_layer(h, w_ref, as_ref, ad_ref, b_ref, od, concat):
        xp = _mm(h, w_ref[...])
        outs = []
        for hd in range(HEADS):
            xp_h = xp[:, hd * od:(hd + 1) * od]
            a_s = as_ref[hd:hd + 1, :]
            a_d = ad_ref[hd:hd + 1, :]
            outs.append(_gat_head(xp_h, a_s, a_d, cpos, cnt))
        if concat:
            o = jnp.concatenate(outs, axis=1)
        else:
            o = (outs[0] + outs[1]) * 0.5
        return o + b_ref[...]

    def elu(z):
        return jnp.where(z > 0.0, z, jnp.exp(jnp.minimum(z, 0.0)) - 1.0)

    h1 = elu(gat_layer(x, w1_ref, as1_ref, ad1_ref, b1_ref, PH, True)
             + _mm(x, wl1_ref[...]) + bl1_ref[...])
    h2 = elu(gat_layer(h1, w2_ref, as2_ref, ad2_ref, b2_ref, PH, True)
             + _mm(h1, wl2_ref[...]) + bl2_ref[...])
    x3 = (gat_layer(h2, w3_ref, as3_ref, ad3_ref, b3_ref, 2 * D_K, False)
          + _mm(h2, wl3_ref[...]) + bl3_ref[...])

    qh = _mm(x, wq_ref[...])
    kh = _mm(k_ref[0], wk_ref[...])
    vh = _mm(v_ref[0], wv_ref[...])
    scale = 1.0 / (D_K ** 0.5)
    os = []
    for hd in range(HEADS):
        q_h = qh[:, hd * D_K:(hd + 1) * D_K] * scale
        k_h = kh[:, hd * D_K:(hd + 1) * D_K]
        v_h = vh[:, hd * D_K:(hd + 1) * D_K]
        logits = _mm_t(q_h, k_h)
        mx = jnp.max(logits, axis=1, keepdims=True)
        ex = jnp.exp(logits - mx)
        a = ex * (1.0 / jnp.sum(ex, axis=1, keepdims=True))
        attn_ref[0, hd] = a
        os.append(_mm(a, v_h))
    o = jnp.concatenate(os, axis=1)

    wfc = wfc_ref[...]
    out = (_mm(x3, wfc[:2 * D_K, :]) + _mm(o, wfc[2 * D_K:, :]) + x)
    mu = jnp.mean(out, axis=1, keepdims=True)
    cen = out - mu
    var = jnp.mean(cen * cen, axis=1, keepdims=True)
    out_ref[0] = cen * jax.lax.rsqrt(var + 1e-6) * gamma_ref[...] + beta_ref[...]


def _tc_forward(C, q, k, v, Wq, Wk, Wv, Wfc,
                W1, as1, ad1, b1, Wl1, bl1,
                W2, as2, ad2, b2, Wl2, bl2,
                W3, as3, ad3, b3, Wl3, bl3,
                gamma, beta, interpret=False):
    full = lambda shape: pl.BlockSpec(shape, lambda b: (0,) * len(shape))
    grid_spec = pl.GridSpec(
        grid=(BS,),
        in_specs=[
            pl.BlockSpec((1, N, N), lambda b: (b, 0, 0)),
            pl.BlockSpec((1, N, D_MODEL), lambda b: (b, 0, 0)),
            pl.BlockSpec((1, N, D_MODEL), lambda b: (b, 0, 0)),
            pl.BlockSpec((1, N, D_MODEL), lambda b: (b, 0, 0)),
            full((D_MODEL, HEADS * D_K)),
            full((D_MODEL, HEADS * D_K)),
            full((D_MODEL, HEADS * D_K)),
            full((4 * D_K, D_MODEL)),
            full((D_MODEL, 2 * PH)),
            full((2, PH)), full((2, PH)), full((1, 2 * PH)),
            full((D_MODEL, 2 * PH)), full((1, 2 * PH)),
            full((2 * PH, 2 * PH)),
            full((2, PH)), full((2, PH)), full((1, 2 * PH)),
            full((2 * PH, 2 * PH)), full((1, 2 * PH)),
            full((2 * PH, 2 * 2 * D_K)),
            full((2, 2 * D_K)), full((2, 2 * D_K)), full((1, 2 * D_K)),
            full((2 * PH, 2 * D_K)), full((1, 2 * D_K)),
            full((1, D_MODEL)), full((1, D_MODEL)),
        ],
        out_specs=[
            pl.BlockSpec((1, N, D_MODEL), lambda b: (b, 0, 0)),
            pl.BlockSpec((1, HEADS, N, N), lambda b: (b, 0, 0, 0)),
        ],
    )
    return pl.pallas_call(
        _tc_body,
        grid_spec=grid_spec,
        out_shape=[
            jax.ShapeDtypeStruct((BS, N, D_MODEL), jnp.float32),
            jax.ShapeDtypeStruct((BS, HEADS, N, N), jnp.float32),
        ],
        interpret=interpret,
    )(C, q, k, v, Wq, Wk, Wv, Wfc,
      W1, as1, ad1, b1.reshape(1, -1), Wl1, bl1.reshape(1, -1),
      W2, as2, ad2, b2.reshape(1, -1), Wl2, bl2.reshape(1, -1),
      W3, as3, ad3, b3.reshape(1, -1), Wl3, bl3.reshape(1, -1),
      gamma.reshape(1, -1), beta.reshape(1, -1))


def kernel(q, k, v, edge_index, Wq, Wk, Wv, Wfc, W1, as1, ad1, b1, Wl1, bl1,
           W2, as2, ad2, b2, Wl2, bl2, W3, as3, ad3, b3, Wl3, bl3,
           gamma, beta):
    src = edge_index[:, 0, :]
    dst = edge_index[:, 1, :]
    C = _build_counts(src, dst)
    out, attn = _tc_forward(C, q, k, v, Wq, Wk, Wv, Wfc,
                            W1, as1, ad1, b1, Wl1, bl1,
                            W2, as2, ad2, b2, Wl2, bl2,
                            W3, as3, ad3, b3, Wl3, bl3,
                            gamma, beta)
    return (out, attn)
